# baseline (speedup 1.0000x reference)
import functools

import jax
import jax.numpy as jnp
from jax import lax
from jax.experimental import pallas as pl
from jax.experimental.pallas import tpu as pltpu
from jax.experimental.pallas import tpu_sc as plsc

_B = 1024
_RANK = 32
_N_ENT = 100000
_TILE_E = 4096
_LANES = 16
_NC = 2
_NS = 16


def _sc_gather_call(entity_emb, rel_emb, heads, rels, tails):
    nw = _NC * _NS
    bpw = _B // nw
    mesh = plsc.VectorSubcoreMesh(core_axis_name="c", subcore_axis_name="s")

    @functools.partial(
        pl.kernel,
        mesh=mesh,
        out_type=(
            jax.ShapeDtypeStruct((_B, _RANK), jnp.float32),
            jax.ShapeDtypeStruct((_B, _RANK), jnp.float32),
        ),
        scratch_types=[
            pltpu.VMEM((bpw,), jnp.int32),
            pltpu.VMEM((bpw,), jnp.int32),
            pltpu.VMEM((bpw,), jnp.int32),
            pltpu.VMEM((bpw, _RANK), jnp.float32),
            pltpu.VMEM((bpw, _RANK), jnp.float32),
            pltpu.VMEM((bpw, _RANK), jnp.float32),
            pltpu.SemaphoreType.DMA,
            pltpu.SemaphoreType.DMA,
            pltpu.SemaphoreType.DMA,
        ],
        compiler_params=pltpu.CompilerParams(use_tc_tiling_on_sc=False),
    )
    def k(ent_hbm, rel_hbm, h_hbm, r_hbm, t_hbm, q_out, te_out,
          hv, rv, tv, lhs_v, rel_v, te_v, sem1, sem2, sem3):
        wid = lax.axis_index("s") * _NC + lax.axis_index("c")
        base = wid * bpw
        pltpu.sync_copy(h_hbm.at[pl.ds(base, bpw)], hv)
        pltpu.sync_copy(r_hbm.at[pl.ds(base, bpw)], rv)
        pltpu.sync_copy(t_hbm.at[pl.ds(base, bpw)], tv)
        c1 = pltpu.async_copy(ent_hbm.at[hv], lhs_v, sem1)
        c2 = pltpu.async_copy(rel_hbm.at[rv], rel_v, sem2)
        c3 = pltpu.async_copy(ent_hbm.at[tv], te_v, sem3)
        c1.wait()
        c2.wait()
        for i in range(bpw):
            for j in range(_RANK // _LANES):
                sl = pl.ds(j * _LANES, _LANES)
                lhs_v[i, sl] = lhs_v[i, sl] * rel_v[i, sl]
        pltpu.sync_copy(lhs_v, q_out.at[pl.ds(base, bpw)])
        c3.wait()
        pltpu.sync_copy(te_v, te_out.at[pl.ds(base, bpw)])

    return k(entity_emb, rel_emb, heads, rels, tails)


_TILE_B = 32


_CHUNK_E = 25088
_NK = -(-_N_ENT // _CHUNK_E)


def _tc_body(q_ref, te_ref, tgt_ref, embt_ref, masked_ref, ranks_ref, cnt_ref):
    k = pl.program_id(1)
    ts = jnp.sum(q_ref[...] * te_ref[...], axis=1, keepdims=True)
    scores = jnp.dot(q_ref[...], embt_ref[...],
                     preferred_element_type=jnp.float32)
    cols = k * _CHUNK_E + lax.broadcasted_iota(
        jnp.int32, (_TILE_B, _CHUNK_E), 1)
    masked = jnp.where(cols == tgt_ref[...], -1000000.0, scores)
    masked_ref[...] = masked
    hit = (masked >= ts) & (cols < _N_ENT)
    cnt = jnp.sum(hit.astype(jnp.float32), axis=1, keepdims=True)

    @pl.when(k == 0)
    def _first():
        cnt_ref[...] = cnt

    @pl.when(k != 0)
    def _rest():
        cnt_ref[...] += cnt

    @pl.when(k == _NK - 1)
    def _last():
        ranks_ref[...] = 1.0 + cnt_ref[...]


def _tc_score_call(q, tgt_e, tgt, embt, interpret=False):
    return pl.pallas_call(
        _tc_body,
        grid=(_B // _TILE_B, _NK),
        in_specs=[
            pl.BlockSpec((_TILE_B, _RANK), lambda i, k: (i, 0)),
            pl.BlockSpec((_TILE_B, _RANK), lambda i, k: (i, 0)),
            pl.BlockSpec((_TILE_B, 1), lambda i, k: (i, 0)),
            pl.BlockSpec((_RANK, _CHUNK_E), lambda i, k: (0, k)),
        ],
        out_specs=[
            pl.BlockSpec((_TILE_B, _CHUNK_E), lambda i, k: (i, k)),
            pl.BlockSpec((_TILE_B, 1), lambda i, k: (i, 0)),
        ],
        out_shape=[
            jax.ShapeDtypeStruct((_B, _N_ENT), jnp.float32),
            jax.ShapeDtypeStruct((_B, 1), jnp.float32),
        ],
        scratch_shapes=[pltpu.VMEM((_TILE_B, 1), jnp.float32)],
        interpret=interpret,
    )(q, tgt_e, tgt, embt)


def kernel(queries, entity_emb, rel_emb):
    heads = queries[:, 0].astype(jnp.int32)
    rels = queries[:, 1].astype(jnp.int32)
    tails = queries[:, 2].astype(jnp.int32)
    q, tgt_e = _sc_gather_call(entity_emb, rel_emb, heads, rels, tails)
    embt = entity_emb.T
    masked, ranks = _tc_score_call(q, tgt_e, tails[:, None], embt)
    return ranks.reshape(_B), masked


# speedup vs baseline: 1.1533x; 1.1533x over previous
import functools

import jax
import jax.numpy as jnp
from jax import lax
from jax.experimental import pallas as pl
from jax.experimental.pallas import tpu as pltpu
from jax.experimental.pallas import tpu_sc as plsc

_B = 1024
_RANK = 32
_N_ENT = 100000
_TILE_E = 4096
_LANES = 16
_NC = 2
_NS = 16


def _sc_gather_call(entity_emb, rel_emb, heads, rels, tails):
    nw = _NC * _NS
    bpw = _B // nw
    mesh = plsc.VectorSubcoreMesh(core_axis_name="c", subcore_axis_name="s")

    @functools.partial(
        pl.kernel,
        mesh=mesh,
        out_type=(
            jax.ShapeDtypeStruct((_B, _RANK), jnp.float32),
            jax.ShapeDtypeStruct((_B, _RANK), jnp.float32),
        ),
        scratch_types=[
            pltpu.VMEM((bpw,), jnp.int32),
            pltpu.VMEM((bpw,), jnp.int32),
            pltpu.VMEM((bpw,), jnp.int32),
            pltpu.VMEM((bpw, _RANK), jnp.float32),
            pltpu.VMEM((bpw, _RANK), jnp.float32),
            pltpu.VMEM((bpw, _RANK), jnp.float32),
            pltpu.SemaphoreType.DMA,
            pltpu.SemaphoreType.DMA,
            pltpu.SemaphoreType.DMA,
        ],
        compiler_params=pltpu.CompilerParams(use_tc_tiling_on_sc=False),
    )
    def k(ent_hbm, rel_hbm, h_hbm, r_hbm, t_hbm, q_out, te_out,
          hv, rv, tv, lhs_v, rel_v, te_v, sem1, sem2, sem3):
        wid = lax.axis_index("s") * _NC + lax.axis_index("c")
        base = wid * bpw
        pltpu.sync_copy(h_hbm.at[pl.ds(base, bpw)], hv)
        pltpu.sync_copy(r_hbm.at[pl.ds(base, bpw)], rv)
        pltpu.sync_copy(t_hbm.at[pl.ds(base, bpw)], tv)
        c1 = pltpu.async_copy(ent_hbm.at[hv], lhs_v, sem1)
        c2 = pltpu.async_copy(rel_hbm.at[rv], rel_v, sem2)
        c3 = pltpu.async_copy(ent_hbm.at[tv], te_v, sem3)
        c1.wait()
        c2.wait()
        for i in range(bpw):
            for j in range(_RANK // _LANES):
                sl = pl.ds(j * _LANES, _LANES)
                lhs_v[i, sl] = lhs_v[i, sl] * rel_v[i, sl]
        pltpu.sync_copy(lhs_v, q_out.at[pl.ds(base, bpw)])
        c3.wait()
        pltpu.sync_copy(te_v, te_out.at[pl.ds(base, bpw)])

    return k(entity_emb, rel_emb, heads, rels, tails)


_TILE_B = 32


_CHUNK_E = 25088
_NK = -(-_N_ENT // _CHUNK_E)


def _tc_body(q_ref, te_ref, tgt_ref, embt_ref, masked_ref, ranks_ref, cnt_ref):
    k = pl.program_id(0)
    i = pl.program_id(1)
    row = pl.ds(i * _TILE_B, _TILE_B)
    ts = jnp.sum(q_ref[...] * te_ref[...], axis=1, keepdims=True)
    scores = jnp.dot(q_ref[...], embt_ref[...],
                     preferred_element_type=jnp.float32)
    cols = k * _CHUNK_E + lax.broadcasted_iota(
        jnp.int32, (_TILE_B, _CHUNK_E), 1)
    masked = jnp.where(cols == tgt_ref[...], -1000000.0, scores)
    masked_ref[...] = masked
    hit = (masked >= ts) & (cols < _N_ENT)
    cnt = jnp.sum(hit.astype(jnp.float32), axis=1, keepdims=True)

    @pl.when(k == 0)
    def _first():
        cnt_ref[row, :] = cnt

    @pl.when(k != 0)
    def _rest():
        cnt_ref[row, :] += cnt

    @pl.when(k == _NK - 1)
    def _last():
        ranks_ref[...] = 1.0 + cnt_ref[row, :]


def _tc_score_call(q, tgt_e, tgt, embt, interpret=False):
    return pl.pallas_call(
        _tc_body,
        grid=(_NK, _B // _TILE_B),
        in_specs=[
            pl.BlockSpec((_TILE_B, _RANK), lambda k, i: (i, 0)),
            pl.BlockSpec((_TILE_B, _RANK), lambda k, i: (i, 0)),
            pl.BlockSpec((_TILE_B, 1), lambda k, i: (i, 0)),
            pl.BlockSpec((_RANK, _CHUNK_E), lambda k, i: (0, k)),
        ],
        out_specs=[
            pl.BlockSpec((_TILE_B, _CHUNK_E), lambda k, i: (i, k)),
            pl.BlockSpec((_TILE_B, 1), lambda k, i: (i, 0)),
        ],
        out_shape=[
            jax.ShapeDtypeStruct((_B, _N_ENT), jnp.float32),
            jax.ShapeDtypeStruct((_B, 1), jnp.float32),
        ],
        scratch_shapes=[pltpu.VMEM((_B, 1), jnp.float32)],
        interpret=interpret,
    )(q, tgt_e, tgt, embt)


def kernel(queries, entity_emb, rel_emb):
    heads = queries[:, 0].astype(jnp.int32)
    rels = queries[:, 1].astype(jnp.int32)
    tails = queries[:, 2].astype(jnp.int32)
    q, tgt_e = _sc_gather_call(entity_emb, rel_emb, heads, rels, tails)
    embt = entity_emb.T
    masked, ranks = _tc_score_call(q, tgt_e, tails[:, None], embt)
    return ranks.reshape(_B), masked


# speedup vs baseline: 1.2550x; 1.0881x over previous
import functools

import jax
import jax.numpy as jnp
from jax import lax
from jax.experimental import pallas as pl
from jax.experimental.pallas import tpu as pltpu
from jax.experimental.pallas import tpu_sc as plsc

_B = 1024
_RANK = 32
_N_ENT = 100000
_TILE_E = 4096
_LANES = 16
_NC = 2
_NS = 16


def _sc_gather_call(entity_emb, rel_emb, heads, rels, tails):
    nw = _NC * _NS
    bpw = _B // nw
    mesh = plsc.VectorSubcoreMesh(core_axis_name="c", subcore_axis_name="s")

    @functools.partial(
        pl.kernel,
        mesh=mesh,
        out_type=(
            jax.ShapeDtypeStruct((_B, _RANK), jnp.float32),
            jax.ShapeDtypeStruct((_B, _RANK), jnp.float32),
        ),
        scratch_types=[
            pltpu.VMEM((bpw,), jnp.int32),
            pltpu.VMEM((bpw,), jnp.int32),
            pltpu.VMEM((bpw,), jnp.int32),
            pltpu.VMEM((bpw, _RANK), jnp.float32),
            pltpu.VMEM((bpw, _RANK), jnp.float32),
            pltpu.VMEM((bpw, _RANK), jnp.float32),
            pltpu.SemaphoreType.DMA,
            pltpu.SemaphoreType.DMA,
            pltpu.SemaphoreType.DMA,
        ],
        compiler_params=pltpu.CompilerParams(use_tc_tiling_on_sc=False),
    )
    def k(ent_hbm, rel_hbm, h_hbm, r_hbm, t_hbm, q_out, te_out,
          hv, rv, tv, lhs_v, rel_v, te_v, sem1, sem2, sem3):
        wid = lax.axis_index("s") * _NC + lax.axis_index("c")
        base = wid * bpw
        pltpu.sync_copy(h_hbm.at[pl.ds(base, bpw)], hv)
        pltpu.sync_copy(r_hbm.at[pl.ds(base, bpw)], rv)
        pltpu.sync_copy(t_hbm.at[pl.ds(base, bpw)], tv)
        c1 = pltpu.async_copy(ent_hbm.at[hv], lhs_v, sem1)
        c2 = pltpu.async_copy(rel_hbm.at[rv], rel_v, sem2)
        c3 = pltpu.async_copy(ent_hbm.at[tv], te_v, sem3)
        c1.wait()
        c2.wait()
        for i in range(bpw):
            for j in range(_RANK // _LANES):
                sl = pl.ds(j * _LANES, _LANES)
                lhs_v[i, sl] = lhs_v[i, sl] * rel_v[i, sl]
        pltpu.sync_copy(lhs_v, q_out.at[pl.ds(base, bpw)])
        c3.wait()
        pltpu.sync_copy(te_v, te_out.at[pl.ds(base, bpw)])

    return k(entity_emb, rel_emb, heads, rels, tails)


_TILE_B = 32


def _tc_body(q_ref, te_ref, tgt_ref, embt_ref, masked_ref, ranks_ref):
    ts = jnp.sum(q_ref[...] * te_ref[...], axis=1, keepdims=True)
    scores = jnp.dot(q_ref[...], embt_ref[...],
                     preferred_element_type=jnp.float32)
    cols = lax.broadcasted_iota(jnp.int32, (_TILE_B, _N_ENT), 1)
    masked = jnp.where(cols == tgt_ref[...], -1000000.0, scores)
    masked_ref[...] = masked
    ranks_ref[...] = 1.0 + jnp.sum(
        (masked >= ts).astype(jnp.float32), axis=1, keepdims=True)


def _tc_score_call(q, tgt_e, tgt, embt, interpret=False):
    return pl.pallas_call(
        _tc_body,
        grid=(_B // _TILE_B,),
        in_specs=[
            pl.BlockSpec((_TILE_B, _RANK), lambda i: (i, 0)),
            pl.BlockSpec((_TILE_B, _RANK), lambda i: (i, 0)),
            pl.BlockSpec((_TILE_B, 1), lambda i: (i, 0)),
            pl.BlockSpec((_RANK, _N_ENT), lambda i: (0, 0)),
        ],
        out_specs=[
            pl.BlockSpec((_TILE_B, _N_ENT), lambda i: (i, 0)),
            pl.BlockSpec((_TILE_B, 1), lambda i: (i, 0)),
        ],
        out_shape=[
            jax.ShapeDtypeStruct((_B, _N_ENT), jnp.float32),
            jax.ShapeDtypeStruct((_B, 1), jnp.float32),
        ],
        compiler_params=pltpu.CompilerParams(
            dimension_semantics=("parallel",)),
        interpret=interpret,
    )(q, tgt_e, tgt, embt)


def kernel(queries, entity_emb, rel_emb):
    heads = queries[:, 0].astype(jnp.int32)
    rels = queries[:, 1].astype(jnp.int32)
    tails = queries[:, 2].astype(jnp.int32)
    q, tgt_e = _sc_gather_call(entity_emb, rel_emb, heads, rels, tails)
    embt = entity_emb.T
    masked, ranks = _tc_score_call(q, tgt_e, tails[:, None], embt)
    return ranks.reshape(_B), masked
